# baseline (speedup 1.0000x reference)
import functools

import jax
import jax.numpy as jnp
from jax import lax
from jax.experimental import pallas as pl
from jax.experimental.pallas import tpu as pltpu
from jax.experimental.pallas import tpu_sc as plsc

B = 4096
S = 200
D = 64
V = 100000

N = B * S
NW = 32
PER_W = N // NW
CHUNK = 128
NCH = PER_W // CHUNK
POS_PERIOD = 3200
L = 16
NBUF = 4
LOOK = 3
NSUB = 8
SUB = CHUNK // NSUB
DEAD_W = 512


def _sc_lookup(seq_flat, mask_flat, item_emb, posi_emb, pos_tab):
    mesh = plsc.VectorSubcoreMesh(core_axis_name="c", subcore_axis_name="s")

    @functools.partial(
        pl.kernel,
        mesh=mesh,
        out_type=jax.ShapeDtypeStruct((N, D), jnp.float32),
        scratch_types=[
            pltpu.VMEM((PER_W,), jnp.int32),
            pltpu.VMEM((PER_W,), jnp.int32),
            pltpu.VMEM((POS_PERIOD,), jnp.int32),
            pltpu.VMEM((S, D), jnp.float32),
        ]
        + [pltpu.VMEM((CHUNK,), jnp.int32) for _ in range(NBUF)]
        + [pltpu.VMEM((CHUNK, D), jnp.float32) for _ in range(NBUF)]
        + [pltpu.SemaphoreType.DMA for _ in range(2 * NBUF)],
        compiler_params=pltpu.CompilerParams(use_tc_tiling_on_sc=False,
                                             needs_layout_passes=False),
    )
    def k(seq_hbm, mask_hbm, item_hbm, posi_hbm, pos_hbm, out_hbm,
          seq_v, mask_v, pos_v, posi_l,
          i0, i1, i2, i3, r0, r1, r2, r3,
          sg0, sg1, sg2, sg3, ss0, ss1, ss2, ss3):
        ii = (i0, i1, i2, i3)
        rows = (r0, r1, r2, r3)
        sg = (sg0, sg1, sg2, sg3)
        ss = (ss0, ss1, ss2, ss3)

        wid = lax.axis_index("s") * 2 + lax.axis_index("c")
        wbase = wid * PER_W
        pltpu.sync_copy(seq_hbm.at[pl.ds(wbase, PER_W)], seq_v)
        pltpu.sync_copy(mask_hbm.at[pl.ds(wbase, PER_W)], mask_v)
        pltpu.sync_copy(pos_hbm, pos_v)
        pltpu.sync_copy(posi_hbm, posi_l)

        def compute_idx(c, b):
            off = c * CHUNK
            for q in range(CHUNK // L):
                sv = seq_v[pl.ds(off + q * L, L)]
                mv = mask_v[pl.ds(off + q * L, L)]
                ii[b][pl.ds(q * L, L)] = jnp.where(mv == 0,
                                                   sv & (DEAD_W - 1), sv)

        def gather_start(b):
            for j in range(NSUB):
                idx = ii[b].at[pl.ds(j * SUB, SUB)]
                dst = rows[b].at[pl.ds(j * SUB, SUB), :]
                pltpu.make_async_copy(item_hbm.at[idx], dst, sg[b]).start()

        def gather_wait(b):
            for j in range(NSUB):
                idx = ii[b].at[pl.ds(j * SUB, SUB)]
                dst = rows[b].at[pl.ds(j * SUB, SUB), :]
                pltpu.make_async_copy(item_hbm.at[idx], dst, sg[b]).wait()

        def store_cp(c, b):
            dst = out_hbm.at[pl.ds(wbase + c * CHUNK, CHUNK), :]
            return pltpu.make_async_copy(rows[b], dst, ss[b])

        def add_posi(c, b):
            iota = lax.iota(jnp.int32, L)
            off = c * CHUNK
            poff = lax.rem(off, POS_PERIOD)
            for g in range(CHUNK // L):
                rv = iota + g * L
                prow = pos_v[pl.ds(poff + g * L, L)]
                mf = mask_v[pl.ds(off + g * L, L)].astype(jnp.float32)

                def col_body(cc, carry):
                    rv2, prow2, mf2 = carry
                    for u in range(4):
                        colv = jnp.full((L,), cc * 4 + u, dtype=jnp.int32)
                        it = plsc.load_gather(rows[b], [rv2, colv])
                        po = plsc.load_gather(posi_l, [prow2, colv])
                        plsc.store_scatter(rows[b], [rv2, colv],
                                           (it + po) * mf2)
                    return carry

                lax.fori_loop(0, D // 4, col_body, (rv, prow, mf))

        for c0 in range(LOOK):
            compute_idx(c0, c0)
            gather_start(c0)

        def outer(i, carry):
            for p in range(NBUF):
                c = i * NBUF + p
                p3 = (p + LOOK) % NBUF
                gather_wait(p)
                add_posi(c, p)
                store_cp(c, p).start()
                c3 = c + LOOK

                @pl.when(c3 < NCH)
                def _():
                    compute_idx(c3, p3)

                @pl.when((c3 < NCH) & (c >= 1))
                def _():
                    store_cp(c - 1, p3).wait()

                @pl.when(c3 < NCH)
                def _():
                    gather_start(p3)
            return carry

        lax.fori_loop(0, NCH // NBUF, outer, 0)
        for p in range(NBUF):
            store_cp(NCH - NBUF + p, p).wait()

    return k(seq_flat, mask_flat, item_emb, posi_emb, pos_tab)


def kernel(seq, mask, item_emb, posi_emb):
    seq_flat = seq.reshape(N)
    mask_flat = mask.reshape(N)
    pos_tab = (jnp.arange(POS_PERIOD, dtype=jnp.int32) % S).astype(jnp.int32)
    out = _sc_lookup(seq_flat, mask_flat, item_emb, posi_emb, pos_tab)
    return out.reshape(B, S, D)


# speedup vs baseline: 2.7789x; 2.7789x over previous
import functools

import jax
import jax.numpy as jnp
from jax import lax
from jax.experimental import pallas as pl
from jax.experimental.pallas import tpu as pltpu
from jax.experimental.pallas import tpu_sc as plsc

B = 4096
S = 200
D = 64
V = 100000

N = B * S
NW = 32
PER_W = N // NW
CHUNK = 128
NCH = PER_W // CHUNK
POS_PERIOD = 3200
L = 16
NBUF = 4
LOOK = 3
NSUB = 8
SUB = CHUNK // NSUB
DEAD_W = 512


def _sc_lookup(seq_flat, mask_flat, item_emb, posi_emb, pos_tab):
    mesh = plsc.VectorSubcoreMesh(core_axis_name="c", subcore_axis_name="s")

    @functools.partial(
        pl.kernel,
        mesh=mesh,
        out_type=jax.ShapeDtypeStruct((N, D), jnp.float32),
        scratch_types=[
            pltpu.VMEM((PER_W,), jnp.int32),
            pltpu.VMEM((PER_W,), jnp.int32),
            pltpu.VMEM((POS_PERIOD,), jnp.int32),
            pltpu.VMEM((S, D), jnp.float32),
        ]
        + [pltpu.VMEM((CHUNK,), jnp.int32) for _ in range(NBUF)]
        + [pltpu.VMEM((CHUNK, D), jnp.float32) for _ in range(NBUF)]
        + [pltpu.SemaphoreType.DMA for _ in range(2 * NBUF)],
        compiler_params=pltpu.CompilerParams(use_tc_tiling_on_sc=False,
                                             needs_layout_passes=False),
    )
    def k(seq_hbm, mask_hbm, item_hbm, posi_hbm, pos_hbm, out_hbm,
          seq_v, mask_v, pos_v, posi_l,
          i0, i1, i2, i3, r0, r1, r2, r3,
          sg0, sg1, sg2, sg3, ss0, ss1, ss2, ss3):
        ii = (i0, i1, i2, i3)
        rows = (r0, r1, r2, r3)
        sg = (sg0, sg1, sg2, sg3)
        ss = (ss0, ss1, ss2, ss3)

        wid = lax.axis_index("s") * 2 + lax.axis_index("c")
        wbase = wid * PER_W
        pltpu.sync_copy(seq_hbm.at[pl.ds(wbase, PER_W)], seq_v)
        pltpu.sync_copy(mask_hbm.at[pl.ds(wbase, PER_W)], mask_v)
        pltpu.sync_copy(pos_hbm, pos_v)
        pltpu.sync_copy(posi_hbm, posi_l)

        def compute_idx(c, b):
            off = c * CHUNK
            for q in range(CHUNK // L):
                sv = seq_v[pl.ds(off + q * L, L)]
                mv = mask_v[pl.ds(off + q * L, L)]
                ii[b][pl.ds(q * L, L)] = jnp.where(mv == 0,
                                                   sv & (DEAD_W - 1), sv)

        def gather_start(b):
            for j in range(NSUB):
                idx = ii[b].at[pl.ds(j * SUB, SUB)]
                dst = rows[b].at[pl.ds(j * SUB, SUB), :]
                pltpu.make_async_copy(item_hbm.at[idx], dst, sg[b]).start()

        def gather_wait(b):
            for j in range(NSUB):
                idx = ii[b].at[pl.ds(j * SUB, SUB)]
                dst = rows[b].at[pl.ds(j * SUB, SUB), :]
                pltpu.make_async_copy(item_hbm.at[idx], dst, sg[b]).wait()

        def store_cp(c, b):
            dst = out_hbm.at[pl.ds(wbase + c * CHUNK, CHUNK), :]
            return pltpu.make_async_copy(rows[b], dst, ss[b])

        def add_posi(c, b):
            off = c * CHUNK

            def grp_body(g, carry):
                gb = g * L
                mv = mask_v[pl.ds(off + gb, L)].astype(jnp.float32)
                for rl in range(L):
                    mfs = mv[rl]
                    r = gb + rl
                    pos = lax.rem(off + r, S)
                    for u in range(D // L):
                        sl = pl.ds(u * L, L)
                        rows[b][r, sl] = (rows[b][r, sl] + posi_l[pos, sl]) * mfs
                return carry

            lax.fori_loop(0, CHUNK // L, grp_body, 0)

        for c0 in range(LOOK):
            compute_idx(c0, c0)
            gather_start(c0)

        def outer(i, carry):
            for p in range(NBUF):
                c = i * NBUF + p
                p3 = (p + LOOK) % NBUF
                gather_wait(p)
                add_posi(c, p)
                store_cp(c, p).start()
                c3 = c + LOOK

                @pl.when(c3 < NCH)
                def _():
                    compute_idx(c3, p3)

                @pl.when((c3 < NCH) & (c >= 1))
                def _():
                    store_cp(c - 1, p3).wait()

                @pl.when(c3 < NCH)
                def _():
                    gather_start(p3)
            return carry

        lax.fori_loop(0, NCH // NBUF, outer, 0)
        for p in range(NBUF):
            store_cp(NCH - NBUF + p, p).wait()

    return k(seq_flat, mask_flat, item_emb, posi_emb, pos_tab)


def kernel(seq, mask, item_emb, posi_emb):
    seq_flat = seq.reshape(N)
    mask_flat = mask.reshape(N)
    pos_tab = (jnp.arange(POS_PERIOD, dtype=jnp.int32) % S).astype(jnp.int32)
    out = _sc_lookup(seq_flat, mask_flat, item_emb, posi_emb, pos_tab)
    return out.reshape(B, S, D)


# speedup vs baseline: 4.1554x; 1.4954x over previous
import functools

import jax
import jax.numpy as jnp
from jax import lax
from jax.experimental import pallas as pl
from jax.experimental.pallas import tpu as pltpu
from jax.experimental.pallas import tpu_sc as plsc

B = 4096
S = 200
D = 64
V = 100000

N = B * S
NW = 32
PER_W = N // NW
CHUNK = 128
NCH = PER_W // CHUNK
POS_PERIOD = 3200
L = 16
NBUF = 4
LOOK = 3
NSUB = 8
SUB = CHUNK // NSUB
DEAD_W = 512


def _sc_lookup(seq_flat, mask_flat, item_emb, posi_emb, pos_tab):
    mesh = plsc.VectorSubcoreMesh(core_axis_name="c", subcore_axis_name="s")

    @functools.partial(
        pl.kernel,
        mesh=mesh,
        out_type=jax.ShapeDtypeStruct((N, D), jnp.float32),
        scratch_types=[
            pltpu.VMEM((PER_W,), jnp.int32),
            pltpu.VMEM((PER_W,), jnp.int32),
            pltpu.VMEM((POS_PERIOD,), jnp.int32),
            pltpu.VMEM((S, D), jnp.float32),
        ]
        + [pltpu.VMEM((CHUNK,), jnp.int32) for _ in range(NBUF)]
        + [pltpu.VMEM((CHUNK, D), jnp.float32) for _ in range(NBUF)]
        + [pltpu.SemaphoreType.DMA for _ in range(2 * NBUF)],
        compiler_params=pltpu.CompilerParams(use_tc_tiling_on_sc=False,
                                             needs_layout_passes=False),
    )
    def k(seq_hbm, mask_hbm, item_hbm, posi_hbm, pos_hbm, out_hbm,
          seq_v, mask_v, pos_v, posi_l,
          i0, i1, i2, i3, r0, r1, r2, r3,
          sg0, sg1, sg2, sg3, ss0, ss1, ss2, ss3):
        ii = (i0, i1, i2, i3)
        rows = (r0, r1, r2, r3)
        sg = (sg0, sg1, sg2, sg3)
        ss = (ss0, ss1, ss2, ss3)

        wid = lax.axis_index("s") * 2 + lax.axis_index("c")
        wbase = wid * PER_W
        pltpu.sync_copy(seq_hbm.at[pl.ds(wbase, PER_W)], seq_v)
        pltpu.sync_copy(mask_hbm.at[pl.ds(wbase, PER_W)], mask_v)
        pltpu.sync_copy(pos_hbm, pos_v)
        pltpu.sync_copy(posi_hbm, posi_l)

        def compute_idx(c, b):
            off = c * CHUNK
            for q in range(CHUNK // L):
                sv = seq_v[pl.ds(off + q * L, L)]
                mv = mask_v[pl.ds(off + q * L, L)]
                ii[b][pl.ds(q * L, L)] = jnp.where(mv == 0,
                                                   sv & (DEAD_W - 1), sv)

        def gather_start(b):
            for j in range(NSUB):
                idx = ii[b].at[pl.ds(j * SUB, SUB)]
                dst = rows[b].at[pl.ds(j * SUB, SUB), :]
                pltpu.make_async_copy(item_hbm.at[idx], dst, sg[b]).start()

        def gather_wait(b):
            for j in range(NSUB):
                idx = ii[b].at[pl.ds(j * SUB, SUB)]
                dst = rows[b].at[pl.ds(j * SUB, SUB), :]
                pltpu.make_async_copy(item_hbm.at[idx], dst, sg[b]).wait()

        def store_cp(c, b):
            dst = out_hbm.at[pl.ds(wbase + c * CHUNK, CHUNK), :]
            return pltpu.make_async_copy(rows[b], dst, ss[b])

        def add_posi(c, b):
            off = c * CHUNK

            def grp_body(g, carry):
                gb = g * L
                mv = mask_v[pl.ds(off + gb, L)].astype(jnp.float32)
                for q4 in range(L // 4):
                    rbase = gb + q4 * 4
                    vals = []
                    for rl in range(4):
                        r = rbase + rl
                        pos = lax.rem(off + r, S)
                        for u in range(D // L):
                            sl = pl.ds(u * L, L)
                            vals.append(rows[b][r, sl] + posi_l[pos, sl])
                    k2 = 0
                    for rl in range(4):
                        mfs = mv[q4 * 4 + rl]
                        r = rbase + rl
                        for u in range(D // L):
                            rows[b][r, pl.ds(u * L, L)] = vals[k2] * mfs
                            k2 += 1
                return carry

            lax.fori_loop(0, CHUNK // L, grp_body, 0)

        for c0 in range(LOOK):
            compute_idx(c0, c0)
            gather_start(c0)

        def outer(i, carry):
            for p in range(NBUF):
                c = i * NBUF + p
                p3 = (p + LOOK) % NBUF
                gather_wait(p)
                add_posi(c, p)
                store_cp(c, p).start()
                c3 = c + LOOK

                @pl.when(c3 < NCH)
                def _():
                    compute_idx(c3, p3)

                @pl.when((c3 < NCH) & (c >= 1))
                def _():
                    store_cp(c - 1, p3).wait()

                @pl.when(c3 < NCH)
                def _():
                    gather_start(p3)
            return carry

        lax.fori_loop(0, NCH // NBUF, outer, 0)
        for p in range(NBUF):
            store_cp(NCH - NBUF + p, p).wait()

    return k(seq_flat, mask_flat, item_emb, posi_emb, pos_tab)


def kernel(seq, mask, item_emb, posi_emb):
    seq_flat = seq.reshape(N)
    mask_flat = mask.reshape(N)
    pos_tab = (jnp.arange(POS_PERIOD, dtype=jnp.int32) % S).astype(jnp.int32)
    out = _sc_lookup(seq_flat, mask_flat, item_emb, posi_emb, pos_tab)
    return out.reshape(B, S, D)


# speedup vs baseline: 4.1560x; 1.0001x over previous
import functools

import jax
import jax.numpy as jnp
from jax import lax
from jax.experimental import pallas as pl
from jax.experimental.pallas import tpu as pltpu
from jax.experimental.pallas import tpu_sc as plsc

B = 4096
S = 200
D = 64
V = 100000

N = B * S
NW = 32
PER_W = N // NW
CHUNK = 128
NCH = PER_W // CHUNK
POS_PERIOD = 3200
L = 16
NBUF = 4
LOOK = 3
NSUB = 16
SUB = CHUNK // NSUB
DEAD_W = 512


def _sc_lookup(seq_flat, mask_flat, item_emb, posi_emb, pos_tab):
    mesh = plsc.VectorSubcoreMesh(core_axis_name="c", subcore_axis_name="s")

    @functools.partial(
        pl.kernel,
        mesh=mesh,
        out_type=jax.ShapeDtypeStruct((N, D), jnp.float32),
        scratch_types=[
            pltpu.VMEM((PER_W,), jnp.int32),
            pltpu.VMEM((PER_W,), jnp.int32),
            pltpu.VMEM((POS_PERIOD,), jnp.int32),
            pltpu.VMEM((S, D), jnp.float32),
        ]
        + [pltpu.VMEM((CHUNK,), jnp.int32) for _ in range(NBUF)]
        + [pltpu.VMEM((CHUNK, D), jnp.float32) for _ in range(NBUF)]
        + [pltpu.SemaphoreType.DMA for _ in range(2 * NBUF)],
        compiler_params=pltpu.CompilerParams(use_tc_tiling_on_sc=False,
                                             needs_layout_passes=False),
    )
    def k(seq_hbm, mask_hbm, item_hbm, posi_hbm, pos_hbm, out_hbm,
          seq_v, mask_v, pos_v, posi_l,
          i0, i1, i2, i3, r0, r1, r2, r3,
          sg0, sg1, sg2, sg3, ss0, ss1, ss2, ss3):
        ii = (i0, i1, i2, i3)
        rows = (r0, r1, r2, r3)
        sg = (sg0, sg1, sg2, sg3)
        ss = (ss0, ss1, ss2, ss3)

        wid = lax.axis_index("s") * 2 + lax.axis_index("c")
        wbase = wid * PER_W
        pltpu.sync_copy(seq_hbm.at[pl.ds(wbase, PER_W)], seq_v)
        pltpu.sync_copy(mask_hbm.at[pl.ds(wbase, PER_W)], mask_v)
        pltpu.sync_copy(pos_hbm, pos_v)
        pltpu.sync_copy(posi_hbm, posi_l)

        def compute_idx(c, b):
            off = c * CHUNK
            for q in range(CHUNK // L):
                sv = seq_v[pl.ds(off + q * L, L)]
                mv = mask_v[pl.ds(off + q * L, L)]
                ii[b][pl.ds(q * L, L)] = jnp.where(mv == 0,
                                                   sv & (DEAD_W - 1), sv)

        def gather_start(b):
            for j in range(NSUB):
                idx = ii[b].at[pl.ds(j * SUB, SUB)]
                dst = rows[b].at[pl.ds(j * SUB, SUB), :]
                pltpu.make_async_copy(item_hbm.at[idx], dst, sg[b]).start()

        def gather_wait(b):
            for j in range(NSUB):
                idx = ii[b].at[pl.ds(j * SUB, SUB)]
                dst = rows[b].at[pl.ds(j * SUB, SUB), :]
                pltpu.make_async_copy(item_hbm.at[idx], dst, sg[b]).wait()

        def store_cp(c, b):
            dst = out_hbm.at[pl.ds(wbase + c * CHUNK, CHUNK), :]
            return pltpu.make_async_copy(rows[b], dst, ss[b])

        def add_posi(c, b):
            off = c * CHUNK

            def grp_body(g, carry):
                gb = g * L
                mv = mask_v[pl.ds(off + gb, L)].astype(jnp.float32)
                for q4 in range(L // 4):
                    rbase = gb + q4 * 4
                    vals = []
                    for rl in range(4):
                        r = rbase + rl
                        pos = lax.rem(off + r, S)
                        for u in range(D // L):
                            sl = pl.ds(u * L, L)
                            vals.append(rows[b][r, sl] + posi_l[pos, sl])
                    k2 = 0
                    for rl in range(4):
                        mfs = mv[q4 * 4 + rl]
                        r = rbase + rl
                        for u in range(D // L):
                            rows[b][r, pl.ds(u * L, L)] = vals[k2] * mfs
                            k2 += 1
                return carry

            lax.fori_loop(0, CHUNK // L, grp_body, 0)

        for c0 in range(LOOK):
            compute_idx(c0, c0)
            gather_start(c0)

        def outer(i, carry):
            for p in range(NBUF):
                c = i * NBUF + p
                p3 = (p + LOOK) % NBUF
                gather_wait(p)
                add_posi(c, p)
                store_cp(c, p).start()
                c3 = c + LOOK

                @pl.when(c3 < NCH)
                def _():
                    compute_idx(c3, p3)

                @pl.when((c3 < NCH) & (c >= 1))
                def _():
                    store_cp(c - 1, p3).wait()

                @pl.when(c3 < NCH)
                def _():
                    gather_start(p3)
            return carry

        lax.fori_loop(0, NCH // NBUF, outer, 0)
        for p in range(NBUF):
            store_cp(NCH - NBUF + p, p).wait()

    return k(seq_flat, mask_flat, item_emb, posi_emb, pos_tab)


def kernel(seq, mask, item_emb, posi_emb):
    seq_flat = seq.reshape(N)
    mask_flat = mask.reshape(N)
    pos_tab = (jnp.arange(POS_PERIOD, dtype=jnp.int32) % S).astype(jnp.int32)
    out = _sc_lookup(seq_flat, mask_flat, item_emb, posi_emb, pos_tab)
    return out.reshape(B, S, D)


# speedup vs baseline: 4.1702x; 1.0034x over previous
import functools

import jax
import jax.numpy as jnp
from jax import lax
from jax.experimental import pallas as pl
from jax.experimental.pallas import tpu as pltpu
from jax.experimental.pallas import tpu_sc as plsc

B = 4096
S = 200
D = 64
V = 100000

N = B * S
NW = 32
PER_W = N // NW
CHUNK = 128
NCH = PER_W // CHUNK
POS_PERIOD = 3200
L = 16
NBUF = 4
LOOK = 3
NSUB = 8
SUB = CHUNK // NSUB
DEAD_W = 512


def _sc_lookup(seq_flat, mask_flat, item_emb, posi_emb, pos_tab):
    mesh = plsc.VectorSubcoreMesh(core_axis_name="c", subcore_axis_name="s")

    @functools.partial(
        pl.kernel,
        mesh=mesh,
        out_type=jax.ShapeDtypeStruct((N, D), jnp.float32),
        scratch_types=[
            pltpu.VMEM((PER_W,), jnp.int32),
            pltpu.VMEM((PER_W,), jnp.int32),
            pltpu.VMEM((POS_PERIOD,), jnp.int32),
            pltpu.VMEM((S, D), jnp.float32),
        ]
        + [pltpu.VMEM((CHUNK,), jnp.int32) for _ in range(NBUF)]
        + [pltpu.VMEM((CHUNK, D), jnp.float32) for _ in range(NBUF)]
        + [pltpu.SemaphoreType.DMA for _ in range(2 * NBUF)],
        compiler_params=pltpu.CompilerParams(use_tc_tiling_on_sc=False,
                                             needs_layout_passes=False),
    )
    def k(seq_hbm, mask_hbm, item_hbm, posi_hbm, pos_hbm, out_hbm,
          seq_v, mask_v, pos_v, posi_l,
          i0, i1, i2, i3, r0, r1, r2, r3,
          sg0, sg1, sg2, sg3, ss0, ss1, ss2, ss3):
        ii = (i0, i1, i2, i3)
        rows = (r0, r1, r2, r3)
        sg = (sg0, sg1, sg2, sg3)
        ss = (ss0, ss1, ss2, ss3)

        wid = lax.axis_index("s") * 2 + lax.axis_index("c")
        wbase = wid * PER_W
        pltpu.sync_copy(seq_hbm.at[pl.ds(wbase, PER_W)], seq_v)
        pltpu.sync_copy(mask_hbm.at[pl.ds(wbase, PER_W)], mask_v)
        pltpu.sync_copy(pos_hbm, pos_v)
        pltpu.sync_copy(posi_hbm, posi_l)

        def compute_idx(c, b):
            off = c * CHUNK
            for q in range(CHUNK // L):
                sv = seq_v[pl.ds(off + q * L, L)]
                mv = mask_v[pl.ds(off + q * L, L)]
                ii[b][pl.ds(q * L, L)] = jnp.where(mv == 0,
                                                   sv & (DEAD_W - 1), sv)

        def gather_start(b):
            for j in range(NSUB):
                idx = ii[b].at[pl.ds(j * SUB, SUB)]
                dst = rows[b].at[pl.ds(j * SUB, SUB), :]
                pltpu.make_async_copy(item_hbm.at[idx], dst, sg[b]).start()

        def gather_wait(b):
            for j in range(NSUB):
                idx = ii[b].at[pl.ds(j * SUB, SUB)]
                dst = rows[b].at[pl.ds(j * SUB, SUB), :]
                pltpu.make_async_copy(item_hbm.at[idx], dst, sg[b]).wait()

        def store_cp(c, b):
            dst = out_hbm.at[pl.ds(wbase + c * CHUNK, CHUNK), :]
            return pltpu.make_async_copy(rows[b], dst, ss[b])

        def add_posi(c, b):
            off = c * CHUNK

            def grp_body(g, carry):
                gb = g * L
                mv = mask_v[pl.ds(off + gb, L)].astype(jnp.float32)
                for q4 in range(L // 4):
                    rbase = gb + q4 * 4
                    vals = []
                    for rl in range(4):
                        r = rbase + rl
                        pos = lax.rem(off + r, S)
                        for u in range(D // L):
                            sl = pl.ds(u * L, L)
                            vals.append(rows[b][r, sl] + posi_l[pos, sl])
                    k2 = 0
                    for rl in range(4):
                        mfs = mv[q4 * 4 + rl]
                        r = rbase + rl
                        for u in range(D // L):
                            rows[b][r, pl.ds(u * L, L)] = vals[k2] * mfs
                            k2 += 1
                return carry

            lax.fori_loop(0, CHUNK // L, grp_body, 0)

        for c0 in range(LOOK):
            compute_idx(c0, c0)
            gather_start(c0)

        def outer(i, carry):
            for p in range(NBUF):
                c = i * NBUF + p
                p3 = (p + LOOK) % NBUF
                gather_wait(p)
                add_posi(c, p)
                store_cp(c, p).start()
                c3 = c + LOOK

                @pl.when(c3 < NCH)
                def _():
                    compute_idx(c3, p3)

                @pl.when((c3 < NCH) & (c >= 1))
                def _():
                    store_cp(c - 1, p3).wait()

                @pl.when(c3 < NCH)
                def _():
                    gather_start(p3)
            return carry

        lax.fori_loop(0, NCH // NBUF, outer, 0)
        for p in range(NBUF):
            store_cp(NCH - NBUF + p, p).wait()

    return k(seq_flat, mask_flat, item_emb, posi_emb, pos_tab)


def kernel(seq, mask, item_emb, posi_emb):
    seq_flat = seq.reshape(N)
    mask_flat = mask.reshape(N)
    pos_tab = (jnp.arange(POS_PERIOD, dtype=jnp.int32) % S).astype(jnp.int32)
    out = _sc_lookup(seq_flat, mask_flat, item_emb, posi_emb, pos_tab)
    return out.reshape(B, S, D)
